# Initial kernel scaffold; baseline (speedup 1.0000x reference)
import jax
import jax.numpy as jnp
from jax.experimental import pallas as pl


def kernel(xh0, edge_index, t, conditions, n_frag_switch, combined_mask, edge_attr, params):
    raise NotImplementedError("write your pallas kernel here")


# speedup vs baseline: 2.2654x; 2.2654x over previous
import functools
import jax
import jax.numpy as jnp
from jax import lax
from jax.experimental import pallas as pl
from jax.experimental.pallas import tpu as pltpu
from jax.experimental.pallas import tpu_sc as plsc

N = 10000
E = 320000
HC = 128
ENF = 16
NG = 16
NP = 10240
EP = 323584
CHG = 158
CHT = 158
TW = 256
PW = 128
BE = 2048
BN = 2048
NW = 32
RPT = NP // 16


def _swish(x):
    return x * jax.nn.sigmoid(x)


@functools.cache
def _sc_kernels():
    mesh = plsc.VectorSubcoreMesh(core_axis_name="c", subcore_axis_name="s")

    @functools.partial(
        pl.kernel,
        mesh=mesh,
        out_type=jax.ShapeDtypeStruct((2 * EP, TW), jnp.float32),
        scratch_types=[
            pltpu.VMEM((CHG, 128), jnp.int32),
            pltpu.VMEM((128, TW), jnp.float32),
            pltpu.SemaphoreType.DMA,
        ],
    )
    def sc_gather(table_hbm, idx_hbm, out_hbm, idx_v, rows_v, sem):
        c = lax.axis_index("c")
        s = lax.axis_index("s")
        wid = s * 2 + c
        pltpu.sync_copy(idx_hbm.at[wid], idx_v)

        def body(j, carry):
            pltpu.async_copy(table_hbm.at[idx_v.at[j]], rows_v, sem).wait()
            row0 = pl.multiple_of((wid * CHG + j) * 128, 128)
            pltpu.sync_copy(rows_v, out_hbm.at[pl.ds(row0, 128)])
            return carry

        lax.fori_loop(0, CHG, body, 0)

    @functools.partial(
        pl.kernel,
        mesh=mesh,
        out_type=jax.ShapeDtypeStruct((2, NP, PW), jnp.float32),
        scratch_types=[
            pltpu.VMEM((CHT, 128), jnp.int32),
            pltpu.VMEM((128, PW), jnp.float32),
            pltpu.VMEM_SHARED((NP, PW), jnp.float32),
            pltpu.SemaphoreType.DMA,
        ],
    )
    def sc_scatter(mpay_hbm, wpay_hbm, idx_hbm, zeros_hbm, out_hbm, idx_v,
                   pay_v, acc_sh, sem):
        c = lax.axis_index("c")
        s = lax.axis_index("s")
        stripe = pl.multiple_of(s * RPT, 8)
        pltpu.sync_copy(zeros_hbm.at[pl.ds(stripe, RPT)],
                        acc_sh.at[pl.ds(stripe, RPT)])
        pltpu.sync_copy(idx_hbm.at[s], idx_v)
        plsc.subcore_barrier()

        @pl.when(c == 0)
        def _():
            def body(j, carry):
                row0 = pl.multiple_of((s * CHT + j) * 128, 128)
                pltpu.sync_copy(mpay_hbm.at[pl.ds(row0, 128)], pay_v)
                pltpu.sync_copy(pay_v, acc_sh.at[idx_v.at[j]], add=True)
                return carry

            lax.fori_loop(0, CHT, body, 0)

        @pl.when(c == 1)
        def _():
            def body(j, carry):
                row0 = pl.multiple_of((s * CHT + j) * 128, 128)
                pltpu.sync_copy(wpay_hbm.at[pl.ds(row0, 128)], pay_v)
                pltpu.sync_copy(pay_v, acc_sh.at[idx_v.at[j]], add=True)
                return carry

            lax.fori_loop(0, CHT, body, 0)

        plsc.subcore_barrier()
        pltpu.sync_copy(acc_sh.at[pl.ds(stripe, RPT)],
                        out_hbm.at[c, pl.ds(stripe, RPT)])

    return sc_gather, sc_scatter


def _enc_body(feat_ref, posp_ref, t_ref, w1_ref, b1_ref, w2_ref, b2_ref,
              ew_ref, eb_ref, out_ref):
    x = feat_ref[...]
    h = _swish(x @ w1_ref[...] + b1_ref[...])
    h = h @ w2_ref[...] + b2_ref[...]
    col = lax.broadcasted_iota(jnp.int32, h.shape, 1)
    h = jnp.where(col == 127, t_ref[0, 0], h)
    h = h @ ew_ref[...] + eb_ref[...]
    out_ref[...] = jnp.concatenate([h, posp_ref[...]], axis=1)


def _edge_core(gs_ref, gd_ref, e, w1a_ref, w1b_ref, w1c_ref, w1d_ref, b1_ref,
               w2_ref, b2_ref, xw_ref, xb_ref, mpay_ref, wpay_ref):
    gs = gs_ref[...]
    gd = gd_ref[...]
    hs = gs[:, :HC]
    hd = gd[:, :HC]
    rel = gs[:, HC:HC + 3] - gd[:, HC:HC + 3]
    d2 = jnp.sum(rel * rel, axis=1, keepdims=True)
    m = _swish(hs @ w1a_ref[...] + hd @ w1b_ref[...] + d2 * w1c_ref[...]
               + e @ w1d_ref[...] + b1_ref[...])
    m = _swish(m @ w2_ref[...] + b2_ref[...])
    coef = m @ xw_ref[...] + xb_ref[...]
    wrel = rel / (jnp.sqrt(d2) + 1.0) * coef
    mpay_ref[...] = m
    pad = jnp.zeros((wrel.shape[0], PW - 3), jnp.float32)
    wpay_ref[...] = jnp.concatenate([wrel, pad], axis=1)


def _edge_body_l0(gs_ref, gd_ref, ea_ref, eew1_ref, eeb1_ref, eew2_ref,
                  eeb2_ref, w1a_ref, w1b_ref, w1c_ref, w1d_ref, b1_ref,
                  w2_ref, b2_ref, xw_ref, xb_ref, mpay_ref, wpay_ref, e_ref):
    e = _swish(ea_ref[...] @ eew1_ref[...] + eeb1_ref[...]) @ eew2_ref[...] \
        + eeb2_ref[...]
    e_ref[...] = e
    _edge_core(gs_ref, gd_ref, e, w1a_ref, w1b_ref, w1c_ref, w1d_ref, b1_ref,
               w2_ref, b2_ref, xw_ref, xb_ref, mpay_ref, wpay_ref)


def _edge_body_l1(gs_ref, gd_ref, e_ref, w1a_ref, w1b_ref, w1c_ref, w1d_ref,
                  b1_ref, w2_ref, b2_ref, xw_ref, xb_ref, mpay_ref, wpay_ref):
    _edge_core(gs_ref, gd_ref, e_ref[...], w1a_ref, w1b_ref, w1c_ref,
               w1d_ref, b1_ref, w2_ref, b2_ref, xw_ref, xb_ref, mpay_ref,
               wpay_ref)


def _node_body(tab_ref, m_ref, w_ref, w1a_ref, w1b_ref, b1_ref, w2_ref,
               b2_ref, out_ref):
    tab = tab_ref[...]
    h = tab[:, :HC]
    u = _swish(h @ w1a_ref[...] + m_ref[0] @ w1b_ref[...] + b1_ref[...])
    h2 = h + u @ w2_ref[...] + b2_ref[...]
    out_ref[...] = jnp.concatenate([h2, tab[:, HC:] + w_ref[0]], axis=1)


def _ro_body(tab_ref, mask_ref, w1_ref, b1_ref, w1g_ref, b1g_ref, w2_ref,
             b2_ref, w2g_ref, b2g_ref, w3_ref, b3_ref, out_ref, acc_ref):
    h = tab_ref[:, :HC]
    g1 = jax.nn.sigmoid(h @ w1g_ref[...] + b1g_ref[...])
    v = _swish((h @ w1_ref[...] + b1_ref[...]) * g1)
    g2 = jax.nn.sigmoid(v @ w2g_ref[...] + b2g_ref[...])
    v = _swish((v @ w2_ref[...] + b2_ref[...]) * g2)
    no = v @ w3_ref[...] + b3_ref[...]
    mask = mask_ref[...]
    oh = (mask == lax.broadcasted_iota(jnp.int32, (mask.shape[0], NG), 1))
    oh = oh.astype(jnp.float32)
    vals = jnp.concatenate([no, jnp.ones_like(no)], axis=1)
    blk = lax.dot_general(oh, vals, (((0,), (0,)), ((), ())))

    @pl.when(pl.program_id(0) == 0)
    def _():
        acc_ref[...] = jnp.zeros_like(acc_ref)

    acc_ref[...] += blk

    @pl.when(pl.program_id(0) == pl.num_programs(0) - 1)
    def _():
        a = acc_ref[...]
        out_ref[...] = a[:, 0:1] / jnp.maximum(a[:, 1:2], 1.0)


def _full(shape):
    nd = len(shape)
    return pl.BlockSpec(shape, lambda b: (0,) * nd)


def _rows(width, nblk=0):
    return pl.BlockSpec((BE, width), lambda b, o=nblk: (b + o, 0))


def _nrows(width):
    return pl.BlockSpec((BN, width), lambda b: (b, 0))


def kernel(xh0, edge_index, t, conditions, n_frag_switch, combined_mask,
           edge_attr, params):
    p = params
    f32 = jnp.float32
    pos = xh0[:, :3]
    feat = xh0[:, 3:]
    featp = jnp.zeros((NP, HC), f32).at[:N].set(feat)
    posp = jnp.zeros((NP, TW - HC), f32).at[:N, :3].set(pos)
    maskp = jnp.full((NP, 1), NG + 7, jnp.int32).at[:N, 0].set(combined_mask)
    src = edge_index[0]
    dst = edge_index[1]
    padi = jnp.zeros((EP - E,), jnp.int32)
    gidx = jnp.concatenate([src, padi, dst, padi]).reshape(NW, CHG, 128)
    sidx = jnp.concatenate(
        [dst, jnp.full((EP - E,), NP - 1, jnp.int32)]).reshape(16, CHT, 128)
    eap = jnp.zeros((EP, ENF), f32).at[:E].set(edge_attr)
    zeros_np = jnp.zeros((NP, PW), f32)
    t2 = t.reshape(1, 1)
    sc_gather, sc_scatter = _sc_kernels()

    def row(v):
        return v.reshape(1, -1)

    enc_w2p = jnp.zeros((256, HC), f32).at[:, :127].set(p['enc_W2'])
    enc_b2p = jnp.zeros((1, HC), f32).at[0, :127].set(p['enc_b2'])
    table = pl.pallas_call(
        _enc_body,
        grid=(NP // BN,),
        in_specs=[_nrows(HC), _nrows(TW - HC), _full((1, 1)),
                  _full((HC, 256)), _full((1, 256)), _full((256, HC)),
                  _full((1, HC)), _full((HC, HC)), _full((1, HC))],
        out_specs=_nrows(TW),
        out_shape=jax.ShapeDtypeStruct((NP, TW), f32),
    )(featp, posp, t2, p['enc_W1'], row(p['enc_b1']), enc_w2p, enc_b2p,
      p['emb_W'], row(p['emb_b']))

    e = None
    for l in range(2):
        ew1 = p['l%d_eW1' % l]
        lw = [ew1[:HC], ew1[HC:2 * HC], ew1[2 * HC:2 * HC + 1],
              ew1[2 * HC + 1:], row(p['l%d_eb1' % l]), p['l%d_eW2' % l],
              row(p['l%d_eb2' % l]), p['l%d_xW' % l],
              p['l%d_xb' % l].reshape(1, 1)]
        lw_specs = [_full((HC, HC)), _full((HC, HC)), _full((1, HC)),
                    _full((ENF, HC)), _full((1, HC)), _full((HC, HC)),
                    _full((1, HC)), _full((HC, 1)), _full((1, 1))]
        g = sc_gather(table, gidx)
        pay_shapes = [jax.ShapeDtypeStruct((EP, PW), f32),
                      jax.ShapeDtypeStruct((EP, PW), f32)]
        if l == 0:
            mpay, wpay, e = pl.pallas_call(
                _edge_body_l0,
                grid=(EP // BE,),
                in_specs=[_rows(TW), _rows(TW, EP // BE), _rows(ENF),
                          _full((ENF, 2 * ENF)), _full((1, 2 * ENF)),
                          _full((2 * ENF, ENF)), _full((1, ENF))] + lw_specs,
                out_specs=[_rows(PW), _rows(PW), _rows(ENF)],
                out_shape=pay_shapes + [jax.ShapeDtypeStruct((EP, ENF), f32)],
            )(g, g, eap, p['ee_W1'], row(p['ee_b1']), p['ee_W2'],
              row(p['ee_b2']), *lw)
        else:
            mpay, wpay = pl.pallas_call(
                _edge_body_l1,
                grid=(EP // BE,),
                in_specs=[_rows(TW), _rows(TW, EP // BE), _rows(ENF)]
                         + lw_specs,
                out_specs=[_rows(PW), _rows(PW)],
                out_shape=pay_shapes,
            )(g, g, e, *lw)
        partials = sc_scatter(mpay, wpay, sidx, zeros_np)
        hw1 = p['l%d_hW1' % l]
        table = pl.pallas_call(
            _node_body,
            grid=(NP // BN,),
            in_specs=[_nrows(TW),
                      pl.BlockSpec((1, BN, PW), lambda b: (0, b, 0)),
                      pl.BlockSpec((1, BN, PW), lambda b: (1, b, 0)),
                      _full((HC, HC)), _full((HC, HC)), _full((1, HC)),
                      _full((HC, HC)), _full((1, HC))],
            out_specs=_nrows(TW),
            out_shape=jax.ShapeDtypeStruct((NP, TW), f32),
        )(table, partials, partials, hw1[:HC], hw1[HC:],
          row(p['l%d_hb1' % l]), p['l%d_hW2' % l], row(p['l%d_hb2' % l]))

    conf = pl.pallas_call(
        _ro_body,
        grid=(NP // BN,),
        in_specs=[_nrows(TW), _nrows(1),
                  _full((HC, HC)), _full((1, HC)), _full((HC, HC)),
                  _full((1, HC)), _full((HC, HC)), _full((1, HC)),
                  _full((HC, HC)), _full((1, HC)), _full((HC, 1)),
                  _full((1, 1))],
        out_specs=_full((NG, 1)),
        out_shape=jax.ShapeDtypeStruct((NG, 1), f32),
        scratch_shapes=[pltpu.VMEM((NG, 2), f32)],
    )(table, maskp, p['ro_W1'], row(p['ro_b1']), p['ro_W1g'],
      row(p['ro_b1g']), p['ro_W2'], row(p['ro_b2']), p['ro_W2g'],
      row(p['ro_b2g']), p['ro_W3'], p['ro_b3'].reshape(1, 1))
    return conf
